# Initial kernel scaffold; baseline (speedup 1.0000x reference)
import jax
import jax.numpy as jnp
from jax.experimental import pallas as pl


def kernel(unit_type_ids, global_positions, unit_positions, unit_lengths, relative_indices, is_first, is_last, is_group_start, has_data, unit_type_table, global_pos_table, unit_pos_table):
    raise NotImplementedError("write your pallas kernel here")


# speedup vs baseline: 8.3479x; 8.3479x over previous
import functools

import jax
import jax.numpy as jnp
from jax import lax
from jax.experimental import pallas as pl
from jax.experimental.pallas import tpu as pltpu
from jax.experimental.pallas import tpu_sc as plsc

NC = 2
NS = 16
LANES = 16
NW = NC * NS

D_UT, D_GP, D_UP, D_SC = 8, 4, 4, 6
D_OUT = D_UT + D_GP + D_UP + D_SC


def _build(N, n_ut, n_gp, n_up, chunk):
    per_w = N // NW
    n_chunks = per_w // chunk
    groups = chunk // LANES
    mesh = plsc.VectorSubcoreMesh(
        core_axis_name="c", subcore_axis_name="s",
        num_cores=NC, num_subcores=NS)

    @functools.partial(
        pl.kernel,
        mesh=mesh,
        compiler_params=pltpu.CompilerParams(needs_layout_passes=False),
        out_type=jax.ShapeDtypeStruct((N * D_OUT,), jnp.float32),
        scratch_types=[
            pltpu.VMEM((n_ut * D_UT,), jnp.float32),
            pltpu.VMEM((n_gp * D_GP,), jnp.float32),
            pltpu.VMEM((n_up * D_UP,), jnp.float32),
            pltpu.VMEM((chunk,), jnp.int32),
            pltpu.VMEM((chunk,), jnp.int32),
            pltpu.VMEM((chunk,), jnp.int32),
            pltpu.VMEM((chunk,), jnp.float32),
            pltpu.VMEM((chunk,), jnp.float32),
            pltpu.VMEM((chunk,), jnp.float32),
            pltpu.VMEM((chunk,), jnp.float32),
            pltpu.VMEM((chunk,), jnp.float32),
            pltpu.VMEM((chunk,), jnp.float32),
            pltpu.VMEM((chunk * D_OUT,), jnp.float32),
            pltpu.SemaphoreType.DMA,
        ],
    )
    def k(ut_hbm, gp_hbm, up_hbm,
          s0_hbm, s1_hbm, s2_hbm, s3_hbm, s4_hbm, s5_hbm,
          ut_tab_hbm, gp_tab_hbm, up_tab_hbm,
          out_hbm,
          ut_tab, gp_tab, up_tab,
          ut_v, gp_v, up_v,
          s0_v, s1_v, s2_v, s3_v, s4_v, s5_v,
          out_v, sem):
        wid = lax.axis_index("s") * NC + lax.axis_index("c")
        wbase = wid * per_w

        pltpu.sync_copy(ut_tab_hbm, ut_tab)
        pltpu.sync_copy(gp_tab_hbm, gp_tab)
        pltpu.sync_copy(up_tab_hbm, up_tab)

        idx_srcs = (ut_hbm, gp_hbm, up_hbm)
        idx_dsts = (ut_v, gp_v, up_v)
        sc_srcs = (s0_hbm, s1_hbm, s2_hbm, s3_hbm, s4_hbm, s5_hbm)
        sc_dsts = (s0_v, s1_v, s2_v, s3_v, s4_v, s5_v)

        for ci in range(n_chunks):
            gbase = wbase + ci * chunk
            cps = []
            for src, dst in zip(idx_srcs + sc_srcs, idx_dsts + sc_dsts):
                cps.append(pltpu.async_copy(
                    src.at[pl.ds(gbase, chunk)], dst, sem))
            for cp in cps:
                cp.wait()

            def body(i, carry):
                base16 = i * LANES
                lane = lax.iota(jnp.int32, LANES)
                row = (lane + base16) * D_OUT
                ut = ut_v[pl.ds(base16, LANES)]
                gp = gp_v[pl.ds(base16, LANES)]
                up = up_v[pl.ds(base16, LANES)]
                for c in range(D_UT):
                    val = plsc.load_gather(ut_tab, [ut * D_UT + c])
                    plsc.store_scatter(out_v, [row + c], val)
                for c in range(D_GP):
                    val = plsc.load_gather(gp_tab, [gp * D_GP + c])
                    plsc.store_scatter(out_v, [row + (D_UT + c)], val)
                for c in range(D_UP):
                    val = plsc.load_gather(up_tab, [up * D_UP + c])
                    plsc.store_scatter(out_v, [row + (D_UT + D_GP + c)], val)
                for j, sref in enumerate((s0_v, s1_v, s2_v, s3_v, s4_v, s5_v)):
                    val = sref[pl.ds(base16, LANES)]
                    plsc.store_scatter(
                        out_v, [row + (D_UT + D_GP + D_UP + j)], val)
                return carry

            lax.fori_loop(0, groups, body, 0)

            pltpu.sync_copy(
                out_v, out_hbm.at[pl.ds(gbase * D_OUT, chunk * D_OUT)])

    return k


def kernel(unit_type_ids, global_positions, unit_positions, unit_lengths,
           relative_indices, is_first, is_last, is_group_start, has_data,
           unit_type_table, global_pos_table, unit_pos_table):
    B, L = unit_type_ids.shape
    N = B * L
    assert N % (NW * LANES) == 0
    per_w = N // NW
    chunk = 3200
    while per_w % chunk != 0:
        chunk //= 2

    k = _build(N, unit_type_table.shape[0], global_pos_table.shape[0],
               unit_pos_table.shape[0], chunk)
    out_flat = k(
        unit_type_ids.reshape(-1), global_positions.reshape(-1),
        unit_positions.reshape(-1),
        unit_lengths.reshape(-1), relative_indices.reshape(-1),
        is_first.reshape(-1), is_last.reshape(-1),
        is_group_start.reshape(-1), has_data.reshape(-1),
        unit_type_table.reshape(-1), global_pos_table.reshape(-1),
        unit_pos_table.reshape(-1))
    return out_flat.reshape(B, L, D_OUT)


# speedup vs baseline: 9.9854x; 1.1962x over previous
import functools

import jax
import jax.numpy as jnp
from jax import lax
from jax.experimental import pallas as pl
from jax.experimental.pallas import tpu as pltpu
from jax.experimental.pallas import tpu_sc as plsc

NC = 2
NS = 16
LANES = 16
NW = NC * NS

D_UT, D_GP, D_UP, D_SC = 8, 4, 4, 6
D_OUT = D_UT + D_GP + D_UP + D_SC


def _build(N, n_ut, n_gp, n_up, chunk, unroll):
    per_w = N // NW
    n_chunks = per_w // chunk
    groups = chunk // LANES
    mesh = plsc.VectorSubcoreMesh(
        core_axis_name="c", subcore_axis_name="s",
        num_cores=NC, num_subcores=NS)

    in_buf = lambda dt: pltpu.VMEM((chunk,), dt)
    scratch = [
        pltpu.VMEM((n_ut * D_UT,), jnp.float32),
        pltpu.VMEM((n_gp * D_GP,), jnp.float32),
        pltpu.VMEM((n_up * D_UP,), jnp.float32),
    ]
    for _ in range(2):
        scratch += [in_buf(jnp.int32)] * 3 + [in_buf(jnp.float32)] * 6
        scratch += [pltpu.VMEM((chunk * D_OUT,), jnp.float32)]
        scratch += [pltpu.SemaphoreType.DMA, pltpu.SemaphoreType.DMA]

    @functools.partial(
        pl.kernel,
        mesh=mesh,
        compiler_params=pltpu.CompilerParams(needs_layout_passes=False),
        out_type=jax.ShapeDtypeStruct((N * D_OUT,), jnp.float32),
        scratch_types=scratch,
    )
    def k(ut_hbm, gp_hbm, up_hbm,
          s0_hbm, s1_hbm, s2_hbm, s3_hbm, s4_hbm, s5_hbm,
          ut_tab_hbm, gp_tab_hbm, up_tab_hbm,
          out_hbm,
          ut_tab, gp_tab, up_tab,
          *bufs):
        sets = [bufs[b * 12:(b + 1) * 12] for b in range(2)]
        wid = lax.axis_index("s") * NC + lax.axis_index("c")
        wbase = wid * per_w

        pltpu.sync_copy(ut_tab_hbm, ut_tab)
        pltpu.sync_copy(gp_tab_hbm, gp_tab)
        pltpu.sync_copy(up_tab_hbm, up_tab)

        srcs = (ut_hbm, gp_hbm, up_hbm,
                s0_hbm, s1_hbm, s2_hbm, s3_hbm, s4_hbm, s5_hbm)

        def start_in(ci, bset):
            gbase = wbase + ci * chunk
            return [pltpu.async_copy(src.at[pl.ds(gbase, chunk)], dst,
                                     bset[10])
                    for src, dst in zip(srcs, bset[:9])]

        lane22 = lax.iota(jnp.int32, LANES) * D_OUT

        def compute(bset):
            ins = bset[:9]
            out_v = bset[9]

            @plsc.parallel_loop(0, groups, 1, unroll=unroll)
            def body(i):
                base16 = i * LANES
                row = lane22 + i * (LANES * D_OUT)
                ut8 = ins[0][pl.ds(base16, LANES)] * D_UT
                gp4 = ins[1][pl.ds(base16, LANES)] * D_GP
                up4 = ins[2][pl.ds(base16, LANES)] * D_UP
                for c in range(D_UT):
                    val = plsc.load_gather(ut_tab, [ut8 + c])
                    plsc.store_scatter(out_v, [row + c], val)
                for c in range(D_GP):
                    val = plsc.load_gather(gp_tab, [gp4 + c])
                    plsc.store_scatter(out_v, [row + (D_UT + c)], val)
                for c in range(D_UP):
                    val = plsc.load_gather(up_tab, [up4 + c])
                    plsc.store_scatter(out_v, [row + (D_UT + D_GP + c)], val)
                for j in range(D_SC):
                    val = ins[3 + j][pl.ds(base16, LANES)]
                    plsc.store_scatter(
                        out_v, [row + (D_UT + D_GP + D_UP + j)], val)

        def out_copy(ci, bset):
            gbase = wbase + ci * chunk
            return pltpu.make_async_copy(
                bset[9],
                out_hbm.at[pl.ds(gbase * D_OUT, chunk * D_OUT)],
                bset[11])

        start_in(0, sets[0])

        @pl.loop(0, n_chunks, step=2)
        def chunk_loop(ci0):
            for b in range(2):
                bset = sets[b]
                ci = ci0 + b

                @pl.when(ci + 1 < n_chunks)
                def _():
                    start_in(ci + 1, sets[1 - b])

                for src, dst in zip(srcs, bset[:9]):
                    pltpu.make_async_copy(
                        src.at[pl.ds(0, chunk)], dst, bset[10]).wait()

                @pl.when(ci >= 2)
                def _():
                    out_copy(ci - 2, bset).wait()

                compute(bset)
                out_copy(ci, bset).start()

        for ci in (n_chunks - 2, n_chunks - 1):
            out_copy(ci, sets[ci % 2]).wait()

    return k


def kernel(unit_type_ids, global_positions, unit_positions, unit_lengths,
           relative_indices, is_first, is_last, is_group_start, has_data,
           unit_type_table, global_pos_table, unit_pos_table):
    B, L = unit_type_ids.shape
    N = B * L
    assert N % (NW * LANES) == 0
    per_w = N // NW
    chunk = 1600
    while per_w % chunk != 0:
        chunk //= 2

    k = _build(N, unit_type_table.shape[0], global_pos_table.shape[0],
               unit_pos_table.shape[0], chunk, unroll=4)
    out_flat = k(
        unit_type_ids.reshape(-1), global_positions.reshape(-1),
        unit_positions.reshape(-1),
        unit_lengths.reshape(-1), relative_indices.reshape(-1),
        is_first.reshape(-1), is_last.reshape(-1),
        is_group_start.reshape(-1), has_data.reshape(-1),
        unit_type_table.reshape(-1), global_pos_table.reshape(-1),
        unit_pos_table.reshape(-1))
    return out_flat.reshape(B, L, D_OUT)


# speedup vs baseline: 95.8656x; 9.6005x over previous
import functools

import jax
import jax.numpy as jnp
from jax import lax
from jax.experimental import pallas as pl
from jax.experimental.pallas import tpu as pltpu
from jax.experimental.pallas import tpu_sc as plsc

NC = 2
NS = 16
LANES = 16
NW = NC * NS

D_UT, D_GP, D_UP, D_SC = 8, 4, 4, 6
D_OUT = D_UT + D_GP + D_UP + D_SC


def _build(N, n_ut, n_gp, n_up, chunk, unroll):
    per_w = N // NW
    n_chunks = per_w // chunk
    groups = chunk // LANES
    mesh = plsc.VectorSubcoreMesh(
        core_axis_name="c", subcore_axis_name="s",
        num_cores=NC, num_subcores=NS)

    in_buf = lambda dt: pltpu.VMEM((chunk,), dt)
    scratch = [
        pltpu.VMEM((n_ut * D_UT,), jnp.float32),
        pltpu.VMEM((n_gp * D_GP,), jnp.float32),
        pltpu.VMEM((n_up * D_UP,), jnp.float32),
    ]
    for _ in range(2):
        scratch += [in_buf(jnp.int32)] * 3 + [in_buf(jnp.float32)] * 6
        scratch += [in_buf(jnp.float32)] * (D_UT + D_GP + D_UP)
        scratch += [pltpu.SemaphoreType.DMA] * 3

    @functools.partial(
        pl.kernel,
        mesh=mesh,
        compiler_params=pltpu.CompilerParams(needs_layout_passes=False),
        out_type=jax.ShapeDtypeStruct((D_OUT * N,), jnp.float32),
        scratch_types=scratch,
    )
    def k(ut_hbm, gp_hbm, up_hbm,
          s0_hbm, s1_hbm, s2_hbm, s3_hbm, s4_hbm, s5_hbm,
          ut_tab_hbm, gp_tab_hbm, up_tab_hbm,
          out_hbm,
          ut_tab, gp_tab, up_tab,
          *bufs):
        per_set = 9 + 16 + 3
        sets = [bufs[b * per_set:(b + 1) * per_set] for b in range(2)]
        wid = lax.axis_index("s") * NC + lax.axis_index("c")
        wbase = wid * per_w

        pltpu.sync_copy(ut_tab_hbm, ut_tab)
        pltpu.sync_copy(gp_tab_hbm, gp_tab)
        pltpu.sync_copy(up_tab_hbm, up_tab)

        srcs = (ut_hbm, gp_hbm, up_hbm,
                s0_hbm, s1_hbm, s2_hbm, s3_hbm, s4_hbm, s5_hbm)

        def in_copies(ci, bset, lo=0, hi=9):
            gbase = wbase + ci * chunk
            return [pltpu.make_async_copy(
                        src.at[pl.ds(gbase, chunk)], dst, bset[25])
                    for src, dst in zip(srcs[lo:hi], bset[lo:hi])]

        def emb_out_copies(ci, bset):
            gbase = wbase + ci * chunk
            return [pltpu.make_async_copy(
                        bset[9 + c],
                        out_hbm.at[pl.ds(c * N + gbase, chunk)], bset[26])
                    for c in range(16)]

        def scalar_out_copies(ci, bset):
            gbase = wbase + ci * chunk
            return [pltpu.make_async_copy(
                        bset[3 + j],
                        out_hbm.at[pl.ds((16 + j) * N + gbase, chunk)],
                        bset[27])
                    for j in range(D_SC)]

        def compute(bset):
            ins = bset[:9]
            emb = bset[9:25]

            @plsc.parallel_loop(0, groups, 1, unroll=unroll)
            def body(i):
                base16 = i * LANES
                sl = pl.ds(base16, LANES)
                ut8 = ins[0][sl] * D_UT
                gp4 = ins[1][sl] * D_GP
                up4 = ins[2][sl] * D_UP
                for c in range(D_UT):
                    emb[c][sl] = plsc.load_gather(ut_tab, [ut8 + c])
                for c in range(D_GP):
                    emb[D_UT + c][sl] = plsc.load_gather(gp_tab, [gp4 + c])
                for c in range(D_UP):
                    emb[D_UT + D_GP + c][sl] = plsc.load_gather(
                        up_tab, [up4 + c])

        for cp in in_copies(0, sets[0]):
            cp.start()

        @pl.loop(0, n_chunks, step=2)
        def chunk_loop(ci0):
            for b in range(2):
                bset = sets[b]
                ci = ci0 + b

                @pl.when(ci + 1 < n_chunks)
                def _():
                    for cp in in_copies(ci + 1, sets[1 - b], 0, 3):
                        cp.start()

                @pl.when(ci >= 1)
                def _():
                    for cp in scalar_out_copies(ci - 1, sets[1 - b]):
                        cp.wait()

                @pl.when(ci + 1 < n_chunks)
                def _():
                    for cp in in_copies(ci + 1, sets[1 - b], 3, 9):
                        cp.start()

                for cp in in_copies(ci, bset):
                    cp.wait()

                @pl.when(ci >= 2)
                def _():
                    for cp in emb_out_copies(ci - 2, bset):
                        cp.wait()

                compute(bset)
                for cp in emb_out_copies(ci, bset):
                    cp.start()
                for cp in scalar_out_copies(ci, bset):
                    cp.start()

        for ci in (n_chunks - 2, n_chunks - 1):
            for cp in emb_out_copies(ci, sets[ci % 2]):
                cp.wait()
        for cp in scalar_out_copies(n_chunks - 1, sets[(n_chunks - 1) % 2]):
            cp.wait()

    return k


def kernel(unit_type_ids, global_positions, unit_positions, unit_lengths,
           relative_indices, is_first, is_last, is_group_start, has_data,
           unit_type_table, global_pos_table, unit_pos_table):
    B, L = unit_type_ids.shape
    N = B * L
    assert B % 128 == 0 and L % 8 == 0
    per_w = N // NW

    def phys(x):
        return (x.T.reshape(L // 8, 8, B // 128, 128)
                .transpose(0, 2, 1, 3).reshape(-1))

    chunk = 2560
    while per_w % chunk != 0 or chunk % LANES != 0:
        chunk //= 2

    k = _build(N, unit_type_table.shape[0], global_pos_table.shape[0],
               unit_pos_table.shape[0], chunk, unroll=8)
    out2 = k(
        phys(unit_type_ids), phys(global_positions), phys(unit_positions),
        phys(unit_lengths), phys(relative_indices),
        phys(is_first), phys(is_last), phys(is_group_start), phys(has_data),
        unit_type_table.reshape(-1), global_pos_table.reshape(-1),
        unit_pos_table.reshape(-1))
    out5 = out2.reshape(D_OUT, L // 8, B // 128, 8, 128)
    return out5.transpose(2, 4, 1, 3, 0).reshape(B, L, D_OUT)


# speedup vs baseline: 106.0481x; 1.1062x over previous
import functools

import jax
import jax.numpy as jnp
from jax import lax
from jax.experimental import pallas as pl
from jax.experimental.pallas import tpu as pltpu
from jax.experimental.pallas import tpu_sc as plsc

NC = 2
NS = 16
LANES = 16
NW = NC * NS

D_UT, D_GP, D_UP, D_SC = 8, 4, 4, 6
D_OUT = D_UT + D_GP + D_UP + D_SC
NSETS = 4


def _build(N, n_ut, n_gp, n_up, chunk, unroll):
    per_w = N // NW
    n_chunks = per_w // chunk
    groups = chunk // LANES
    mesh = plsc.VectorSubcoreMesh(
        core_axis_name="c", subcore_axis_name="s",
        num_cores=NC, num_subcores=NS)

    in_buf = lambda dt: pltpu.VMEM((chunk,), dt)
    scratch = [
        pltpu.VMEM((n_ut * D_UT,), jnp.float32),
        pltpu.VMEM((n_gp * D_GP,), jnp.float32),
        pltpu.VMEM((n_up * D_UP,), jnp.float32),
    ]
    for _ in range(NSETS):
        scratch += [in_buf(jnp.int32)] * 3 + [in_buf(jnp.float32)] * 6
        scratch += [in_buf(jnp.float32)] * (D_UT + D_GP + D_UP)
        scratch += [pltpu.SemaphoreType.DMA] * 3

    @functools.partial(
        pl.kernel,
        mesh=mesh,
        compiler_params=pltpu.CompilerParams(needs_layout_passes=False),
        out_type=jax.ShapeDtypeStruct((D_OUT * N,), jnp.float32),
        scratch_types=scratch,
    )
    def k(ut_hbm, gp_hbm, up_hbm,
          s0_hbm, s1_hbm, s2_hbm, s3_hbm, s4_hbm, s5_hbm,
          ut_tab_hbm, gp_tab_hbm, up_tab_hbm,
          out_hbm,
          ut_tab, gp_tab, up_tab,
          *bufs):
        per_set = 9 + 16 + 3
        sets = [bufs[b * per_set:(b + 1) * per_set] for b in range(NSETS)]
        wid = lax.axis_index("s") * NC + lax.axis_index("c")
        wbase = wid * per_w

        pltpu.sync_copy(ut_tab_hbm, ut_tab)
        pltpu.sync_copy(gp_tab_hbm, gp_tab)
        pltpu.sync_copy(up_tab_hbm, up_tab)

        srcs = (ut_hbm, gp_hbm, up_hbm,
                s0_hbm, s1_hbm, s2_hbm, s3_hbm, s4_hbm, s5_hbm)

        def in_copies(ci, bset, lo=0, hi=9):
            gbase = wbase + ci * chunk
            return [pltpu.make_async_copy(
                        src.at[pl.ds(gbase, chunk)], dst, bset[25])
                    for src, dst in zip(srcs[lo:hi], bset[lo:hi])]

        def emb_out_copies(ci, bset):
            gbase = wbase + ci * chunk
            return [pltpu.make_async_copy(
                        bset[9 + c],
                        out_hbm.at[pl.ds(c * N + gbase, chunk)], bset[26])
                    for c in range(16)]

        def scalar_out_copies(ci, bset):
            gbase = wbase + ci * chunk
            return [pltpu.make_async_copy(
                        bset[3 + j],
                        out_hbm.at[pl.ds((16 + j) * N + gbase, chunk)],
                        bset[27])
                    for j in range(D_SC)]

        def compute(bset):
            ins = bset[:9]
            emb = bset[9:25]

            @plsc.parallel_loop(0, groups, 1, unroll=unroll)
            def body(i):
                base16 = i * LANES
                sl = pl.ds(base16, LANES)
                ut8 = ins[0][sl] * D_UT
                gp4 = ins[1][sl] * D_GP
                up4 = ins[2][sl] * D_UP
                for c in range(D_UT):
                    emb[c][sl] = plsc.load_gather(ut_tab, [ut8 + c])
                for c in range(D_GP):
                    emb[D_UT + c][sl] = plsc.load_gather(gp_tab, [gp4 + c])
                for c in range(D_UP):
                    emb[D_UT + D_GP + c][sl] = plsc.load_gather(
                        up_tab, [up4 + c])

        for ci in range(2):
            for cp in in_copies(ci, sets[ci]):
                cp.start()

        @pl.loop(0, n_chunks, step=NSETS)
        def chunk_loop(ci0):
            for b in range(NSETS):
                bset = sets[b]
                ci = ci0 + b
                nset = sets[(b + 2) % NSETS]

                @pl.when(ci + 2 < n_chunks)
                def _():
                    @pl.when(ci >= 2)
                    def _():
                        for cp in scalar_out_copies(ci - 2, nset):
                            cp.wait()
                    for cp in in_copies(ci + 2, nset):
                        cp.start()

                for cp in in_copies(ci, bset):
                    cp.wait()

                @pl.when(ci >= NSETS)
                def _():
                    for cp in emb_out_copies(ci - NSETS, bset):
                        cp.wait()

                compute(bset)
                for cp in emb_out_copies(ci, bset):
                    cp.start()
                for cp in scalar_out_copies(ci, bset):
                    cp.start()

        for ci in range(max(0, n_chunks - NSETS), n_chunks):
            for cp in emb_out_copies(ci, sets[ci % NSETS]):
                cp.wait()
        for ci in range(max(0, n_chunks - 4), n_chunks):
            for cp in scalar_out_copies(ci, sets[ci % NSETS]):
                cp.wait()

    return k


def kernel(unit_type_ids, global_positions, unit_positions, unit_lengths,
           relative_indices, is_first, is_last, is_group_start, has_data,
           unit_type_table, global_pos_table, unit_pos_table):
    B, L = unit_type_ids.shape
    N = B * L
    assert B % 128 == 0 and L % 8 == 0
    per_w = N // NW

    def phys(x):
        return (x.T.reshape(L // 8, 8, B // 128, 128)
                .transpose(0, 2, 1, 3).reshape(-1))

    chunk = 1280
    while per_w % (chunk * NSETS) != 0 or chunk % LANES != 0:
        chunk //= 2

    k = _build(N, unit_type_table.shape[0], global_pos_table.shape[0],
               unit_pos_table.shape[0], chunk, unroll=8)
    out2 = k(
        phys(unit_type_ids), phys(global_positions), phys(unit_positions),
        phys(unit_lengths), phys(relative_indices),
        phys(is_first), phys(is_last), phys(is_group_start), phys(has_data),
        unit_type_table.reshape(-1), global_pos_table.reshape(-1),
        unit_pos_table.reshape(-1))
    out5 = out2.reshape(D_OUT, L // 8, B // 128, 8, 128)
    return out5.transpose(2, 4, 1, 3, 0).reshape(B, L, D_OUT)
